# baseline (speedup 1.0000x reference)
import functools

import jax
import jax.numpy as jnp
from jax import lax
from jax.experimental import pallas as pl
from jax.experimental.pallas import tpu as pltpu
from jax.experimental.pallas import tpu_sc as plsc

TOPK = 32
EPS = 1e-7
NC, NS, L = 2, 16, 16
NW = NC * NS
NBINS = 256
HPAD = 17
HSIZE = NBINS * HPAD


@functools.lru_cache(maxsize=None)
def _build(B, T, nsc):
    assert nsc % NW == 0 and T % (L * 8) == 0
    rpw = nsc // NW
    n_vecs = T // L
    mesh = plsc.VectorSubcoreMesh(core_axis_name="c", subcore_axis_name="s")

    @functools.partial(
        pl.kernel,
        out_type=jax.ShapeDtypeStruct((B, T), jnp.float32),
        mesh=mesh,
        compiler_params=pltpu.CompilerParams(
            needs_layout_passes=False, skip_device_barrier=True),
        scratch_types=[
            pltpu.VMEM((2 * T,), jnp.float32),
            pltpu.VMEM((HSIZE,), jnp.int32),
            pltpu.VMEM((NBINS,), jnp.int32),
            pltpu.VMEM((3 * L,), jnp.int32),
            pltpu.SemaphoreType.DMA((2,)),
            pltpu.SemaphoreType.DMA((2,)),
        ],
    )
    def sc_kernel(x_hbm, o_hbm, row2_v, hist_v, tot_v, tmp_v, isem, osem):
        wid = lax.axis_index("s") * NC + lax.axis_index("c")
        base_row = wid * rpw
        lanes = lax.iota(jnp.int32, L)
        ones = jnp.full((L,), 1, jnp.int32)

        pltpu.make_async_copy(x_hbm.at[base_row], row2_v.at[pl.ds(0, T)],
                              isem.at[0]).start()

        def do_row(r, _):
            p = r & 1
            q = 1 - p
            rv = row2_v.at[pl.ds(p * T, T)]
            qv = row2_v.at[pl.ds(q * T, T)]
            pltpu.make_async_copy(x_hbm.at[base_row + r], rv,
                                  isem.at[p]).wait()

            shifts = (23, 15, 7, 0)
            widths = (8, 8, 8, 7)
            path_v = jnp.zeros((L,), jnp.int32)
            kk = jnp.full((L,), TOPK, jnp.int32)
            pmask = 0
            for lvl in range(4):
                shift = shifts[lvl]
                fmask = (1 << widths[lvl]) - 1

                @plsc.parallel_loop(0, HSIZE // L, step=1, unroll=4)
                def _(i):
                    hist_v[pl.ds(i * L, L)] = jnp.zeros((L,), jnp.int32)

                pb = path_v
                pm = pmask

                if lvl == 0:
                    @plsc.parallel_loop(0, n_vecs, step=1, unroll=16)
                    def _(i):
                        b = plsc.bitcast(rv[pl.ds(i * L, L)], jnp.int32)
                        idx = (b >> shift) * HPAD + lanes
                        plsc.addupdate_scatter(hist_v, [idx], ones)
                else:
                    @plsc.parallel_loop(0, n_vecs, step=1, unroll=16)
                    def _(i):
                        b = plsc.bitcast(rv[pl.ds(i * L, L)], jnp.int32)
                        ok = (b & pm) == pb
                        idx = ((b >> shift) & fmask) * HPAD + lanes
                        plsc.addupdate_scatter(hist_v, [idx], ones, mask=ok)

                @plsc.parallel_loop(0, NBINS // L, step=1, unroll=2)
                def _(bv):
                    base = bv * L
                    acc = jnp.zeros((L,), jnp.int32)
                    for lane in range(L):
                        gidx = (base + lanes) * HPAD + lane
                        acc = acc + plsc.load_gather(hist_v, [gidx])
                    tot_v[pl.ds(base, L)] = acc

                g = jnp.zeros((L,), jnp.int32)
                for i in range(L):
                    g = g + plsc.load_gather(tot_v, [lanes * L + i])
                pre_g = plsc.cumsum(g)
                tmp_v[pl.ds(0, L)] = pre_g
                total = plsc.load_gather(
                    tmp_v, [jnp.full((L,), L - 1, jnp.int32)])
                sg = total - pre_g + g
                gsel = plsc.all_reduce_population_count(sg >= kk) - 1
                tmp_v[pl.ds(L, L)] = sg
                tmp_v[pl.ds(2 * L, L)] = g
                sat_g = plsc.load_gather(tmp_v, [gsel + L])
                g_at = plsc.load_gather(tmp_v, [gsel + 2 * L])
                above = sat_g - g_at
                tvec = plsc.load_gather(tot_v, [gsel * L + lanes])
                pre_t = plsc.cumsum(tvec)
                st = above + g_at - pre_t + tvec
                lstar = plsc.all_reduce_population_count(st >= kk) - 1
                bstar = gsel * L + lstar
                tmp_v[pl.ds(0, L)] = st
                s_at = plsc.load_gather(tmp_v, [lstar])
                c_at = plsc.load_gather(tot_v, [bstar])
                kk = kk - (s_at - c_at)
                path_v = path_v | (bstar << shift)
                pmask = pmask | (fmask << shift)

            delta_v = plsc.bitcast(path_v, jnp.float32) + EPS

            @pl.when(r >= 1)
            def _():
                pltpu.make_async_copy(qv,
                                      o_hbm.at[base_row + r - 1],
                                      osem.at[q]).wait()

            @pl.when(r + 1 < rpw)
            def _():
                pltpu.make_async_copy(x_hbm.at[base_row + r + 1],
                                      qv, isem.at[q]).start()

            @plsc.parallel_loop(0, n_vecs, step=1, unroll=16,
                                carry=jnp.zeros((L,), jnp.float32))
            def acc(i, a):
                x = rv[pl.ds(i * L, L)]
                return a + jnp.maximum(x - delta_v, 0.0)

            s_v = jnp.full((L,), jnp.sum(acc) + EPS, jnp.float32)
            rs = jnp.full((L,), 1.0, jnp.float32) / s_v

            @plsc.parallel_loop(0, n_vecs, step=1, unroll=16)
            def _(i):
                x = rv[pl.ds(i * L, L)]
                rv[pl.ds(i * L, L)] = jnp.maximum(x - delta_v, 0.0) * rs

            pltpu.make_async_copy(rv, o_hbm.at[base_row + r],
                                  osem.at[p]).start()
            return 0

        lax.fori_loop(0, rpw, do_row, 0, unroll=False)
        lp = (rpw - 1) & 1
        pltpu.make_async_copy(row2_v.at[pl.ds(lp * T, T)],
                              o_hbm.at[base_row + rpw - 1],
                              osem.at[lp]).wait()

    return sc_kernel


@functools.lru_cache(maxsize=None)
def _build_tc(B, T, row0, nrows):
    bs = 8
    assert nrows % bs == 0

    def body(x_ref, o_ref):
        x = x_ref[...]
        b = lax.bitcast_convert_type(x, jnp.int32)

        def it(i, lo):
            cand = lo | (jnp.int32(1) << (29 - i))
            cnt = jnp.sum(jnp.where(b >= cand, 1, 0), axis=1, keepdims=True)
            return jnp.where(cnt >= TOPK, cand, lo)

        lo = lax.fori_loop(0, 30, it, jnp.zeros((bs, 1), jnp.int32))
        delta = lax.bitcast_convert_type(lo, jnp.float32) + EPS
        w = jnp.maximum(x - delta, 0.0)
        s = jnp.sum(w, axis=1, keepdims=True) + EPS
        o_ref[...] = w / s

    return pl.pallas_call(
        body,
        grid=(nrows // bs,),
        in_specs=[pl.BlockSpec((bs, T), lambda i: (row0 // bs + i, 0))],
        out_specs=pl.BlockSpec((bs, T), lambda i: (i, 0)),
        out_shape=jax.ShapeDtypeStruct((nrows, T), jnp.float32),
    )


SC_ROWS = 96


def kernel(attn_s):
    B, T = attn_s.shape
    if T <= TOPK:
        return attn_s
    nsc = SC_ROWS if B == 128 else B - (B % NW)
    if nsc <= 0:
        return _build_tc(B, T, 0, B)(attn_s)
    sc_out = _build(B, T, nsc)(attn_s)
    if nsc == B:
        return sc_out
    tc_out = _build_tc(B, T, nsc, B - nsc)(attn_s)
    return lax.dynamic_update_slice(sc_out, tc_out, (nsc, 0))


# speedup vs baseline: 1.0047x; 1.0047x over previous
import functools

import jax
import jax.numpy as jnp
from jax import lax
from jax.experimental import pallas as pl
from jax.experimental.pallas import tpu as pltpu
from jax.experimental.pallas import tpu_sc as plsc

TOPK = 32
EPS = 1e-7
NC, NS, L = 2, 16, 16
NW = NC * NS
NBINS = 256
HPAD = 17
HSIZE = NBINS * HPAD


@functools.lru_cache(maxsize=None)
def _build(B, T, nsc):
    assert nsc % NW == 0 and T % (L * 8) == 0
    rpw = nsc // NW
    n_vecs = T // L
    mesh = plsc.VectorSubcoreMesh(core_axis_name="c", subcore_axis_name="s")

    @functools.partial(
        pl.kernel,
        out_type=jax.ShapeDtypeStruct((B, T), jnp.float32),
        mesh=mesh,
        compiler_params=pltpu.CompilerParams(
            needs_layout_passes=False, skip_device_barrier=True),
        scratch_types=[
            pltpu.VMEM((2 * T,), jnp.float32),
            pltpu.VMEM((HSIZE,), jnp.int32),
            pltpu.VMEM((NBINS,), jnp.int32),
            pltpu.VMEM((3 * L,), jnp.int32),
            pltpu.SemaphoreType.DMA((2,)),
            pltpu.SemaphoreType.DMA((2,)),
        ],
    )
    def sc_kernel(x_hbm, o_hbm, row2_v, hist_v, tot_v, tmp_v, isem, osem):
        wid = lax.axis_index("s") * NC + lax.axis_index("c")
        base_row = wid * rpw
        lanes = lax.iota(jnp.int32, L)
        ones = jnp.full((L,), 1, jnp.int32)

        pltpu.make_async_copy(x_hbm.at[base_row], row2_v.at[pl.ds(0, T)],
                              isem.at[0]).start()

        def do_row(r, _):
            p = r & 1
            q = 1 - p
            rv = row2_v.at[pl.ds(p * T, T)]
            qv = row2_v.at[pl.ds(q * T, T)]
            pltpu.make_async_copy(x_hbm.at[base_row + r], rv,
                                  isem.at[p]).wait()

            shifts = (23, 15, 7, 0)
            widths = (8, 8, 8, 7)
            path_v = jnp.zeros((L,), jnp.int32)
            kk = jnp.full((L,), TOPK, jnp.int32)
            pmask = 0
            for lvl in range(4):
                shift = shifts[lvl]
                fmask = (1 << widths[lvl]) - 1

                @plsc.parallel_loop(0, HSIZE // L, step=1, unroll=8)
                def _(i):
                    hist_v[pl.ds(i * L, L)] = jnp.zeros((L,), jnp.int32)

                pb = path_v
                pm = pmask

                if lvl == 0:
                    @plsc.parallel_loop(0, n_vecs, step=1, unroll=16)
                    def _(i):
                        b = plsc.bitcast(rv[pl.ds(i * L, L)], jnp.int32)
                        idx = (b >> shift) * HPAD + lanes
                        plsc.addupdate_scatter(hist_v, [idx], ones)
                else:
                    @plsc.parallel_loop(0, n_vecs, step=1, unroll=16)
                    def _(i):
                        b = plsc.bitcast(rv[pl.ds(i * L, L)], jnp.int32)
                        ok = (b & pm) == pb
                        idx = ((b >> shift) & fmask) * HPAD + lanes
                        plsc.addupdate_scatter(hist_v, [idx], ones, mask=ok)

                @plsc.parallel_loop(0, NBINS // L, step=1, unroll=4)
                def _(bv):
                    base = bv * L
                    acc = jnp.zeros((L,), jnp.int32)
                    for lane in range(L):
                        gidx = (base + lanes) * HPAD + lane
                        acc = acc + plsc.load_gather(hist_v, [gidx])
                    tot_v[pl.ds(base, L)] = acc

                g = jnp.zeros((L,), jnp.int32)
                for i in range(L):
                    g = g + plsc.load_gather(tot_v, [lanes * L + i])
                pre_g = plsc.cumsum(g)
                tmp_v[pl.ds(0, L)] = pre_g
                total = plsc.load_gather(
                    tmp_v, [jnp.full((L,), L - 1, jnp.int32)])
                sg = total - pre_g + g
                gsel = plsc.all_reduce_population_count(sg >= kk) - 1
                tmp_v[pl.ds(L, L)] = sg
                tmp_v[pl.ds(2 * L, L)] = g
                sat_g = plsc.load_gather(tmp_v, [gsel + L])
                g_at = plsc.load_gather(tmp_v, [gsel + 2 * L])
                above = sat_g - g_at
                tvec = plsc.load_gather(tot_v, [gsel * L + lanes])
                pre_t = plsc.cumsum(tvec)
                st = above + g_at - pre_t + tvec
                lstar = plsc.all_reduce_population_count(st >= kk) - 1
                bstar = gsel * L + lstar
                tmp_v[pl.ds(0, L)] = st
                s_at = plsc.load_gather(tmp_v, [lstar])
                c_at = plsc.load_gather(tot_v, [bstar])
                kk = kk - (s_at - c_at)
                path_v = path_v | (bstar << shift)
                pmask = pmask | (fmask << shift)

            delta_v = plsc.bitcast(path_v, jnp.float32) + EPS

            @pl.when(r >= 1)
            def _():
                pltpu.make_async_copy(qv,
                                      o_hbm.at[base_row + r - 1],
                                      osem.at[q]).wait()

            @pl.when(r + 1 < rpw)
            def _():
                pltpu.make_async_copy(x_hbm.at[base_row + r + 1],
                                      qv, isem.at[q]).start()

            @plsc.parallel_loop(0, n_vecs, step=1, unroll=16,
                                carry=jnp.zeros((L,), jnp.float32))
            def acc(i, a):
                x = rv[pl.ds(i * L, L)]
                return a + jnp.maximum(x - delta_v, 0.0)

            s_v = jnp.full((L,), jnp.sum(acc) + EPS, jnp.float32)
            rs = jnp.full((L,), 1.0, jnp.float32) / s_v

            @plsc.parallel_loop(0, n_vecs, step=1, unroll=16)
            def _(i):
                x = rv[pl.ds(i * L, L)]
                rv[pl.ds(i * L, L)] = jnp.maximum(x - delta_v, 0.0) * rs

            pltpu.make_async_copy(rv, o_hbm.at[base_row + r],
                                  osem.at[p]).start()
            return 0

        lax.fori_loop(0, rpw, do_row, 0, unroll=False)
        lp = (rpw - 1) & 1
        pltpu.make_async_copy(row2_v.at[pl.ds(lp * T, T)],
                              o_hbm.at[base_row + rpw - 1],
                              osem.at[lp]).wait()

    return sc_kernel


@functools.lru_cache(maxsize=None)
def _build_tc(B, T, row0, nrows):
    bs = 8
    assert nrows % bs == 0

    def body(x_ref, o_ref):
        x = x_ref[...]
        b = lax.bitcast_convert_type(x, jnp.int32)

        def it(i, lo):
            cand = lo | (jnp.int32(1) << (29 - i))
            cnt = jnp.sum(jnp.where(b >= cand, 1, 0), axis=1, keepdims=True)
            return jnp.where(cnt >= TOPK, cand, lo)

        lo = lax.fori_loop(0, 30, it, jnp.zeros((bs, 1), jnp.int32))
        delta = lax.bitcast_convert_type(lo, jnp.float32) + EPS
        w = jnp.maximum(x - delta, 0.0)
        s = jnp.sum(w, axis=1, keepdims=True) + EPS
        o_ref[...] = w / s

    return pl.pallas_call(
        body,
        grid=(nrows // bs,),
        in_specs=[pl.BlockSpec((bs, T), lambda i: (row0 // bs + i, 0))],
        out_specs=pl.BlockSpec((bs, T), lambda i: (i, 0)),
        out_shape=jax.ShapeDtypeStruct((nrows, T), jnp.float32),
    )


SC_ROWS = 96


def kernel(attn_s):
    B, T = attn_s.shape
    if T <= TOPK:
        return attn_s
    nsc = SC_ROWS if B == 128 else B - (B % NW)
    if nsc <= 0:
        return _build_tc(B, T, 0, B)(attn_s)
    sc_out = _build(B, T, nsc)(attn_s)
    if nsc == B:
        return sc_out
    tc_out = _build_tc(B, T, nsc, B - nsc)(attn_s)
    return lax.dynamic_update_slice(sc_out, tc_out, (nsc, 0))
